# speedup vs baseline: 22.6086x; 22.6086x over previous
import functools

import jax
import jax.numpy as jnp
from jax import lax
from jax.experimental import pallas as pl
from jax.experimental.pallas import tpu as pltpu
from jax.experimental.pallas import tpu_sc as plsc

N = 10000
D = 128
E = 320000
NC = 2
NS = 16
NW = NC * NS
K = 128
CH = -(-(E // NW) // K)
EPW = CH * K
EPAD = EPW * NW
ACC_R = 10240
RPT = ACC_R // NS

_mesh = plsc.VectorSubcoreMesh(core_axis_name="c", subcore_axis_name="s")


@functools.partial(
    pl.kernel,
    mesh=_mesh,
    out_type=jax.ShapeDtypeStruct((NC, ACC_R), jnp.float32),
    scratch_types=[
        pltpu.VMEM((CH, K), jnp.int32),
        pltpu.VMEM((K,), jnp.float32),
        pltpu.VMEM_SHARED((ACC_R,), jnp.float32),
    ],
)
def _sc_degree(dstp_hbm, ones_hbm, zeros1_hbm, out_hbm, dst_v, ones_v, dacc):
    cid = lax.axis_index("c")
    sid = lax.axis_index("s")
    wid = sid * NC + cid
    pltpu.sync_copy(zeros1_hbm, dacc.at[pl.ds(sid * RPT, RPT)])
    pltpu.sync_copy(dstp_hbm.at[wid], dst_v)
    pltpu.sync_copy(ones_hbm, ones_v)
    plsc.subcore_barrier()

    def body(j, carry):
        pltpu.sync_copy(ones_v, dacc.at[dst_v.at[j]], add=True)
        return carry

    lax.fori_loop(0, CH, body, 0)
    plsc.subcore_barrier()
    pltpu.sync_copy(dacc.at[pl.ds(sid * RPT, RPT)],
                    out_hbm.at[cid, pl.ds(sid * RPT, RPT)])


@functools.partial(
    pl.kernel,
    mesh=_mesh,
    out_type=jax.ShapeDtypeStruct((NC, ACC_R, D), jnp.float32),
    scratch_types=[
        pltpu.VMEM((CH, K), jnp.int32),
        pltpu.VMEM((CH, K), jnp.int32),
        pltpu.VMEM((K, D), jnp.float32),
        pltpu.VMEM_SHARED((ACC_R, D), jnp.float32),
        pltpu.SemaphoreType.DMA,
    ],
)
def _sc_scatter(g_hbm, srcp_hbm, dstp_hbm, zeros_hbm, out_hbm,
                src_v, dst_v, rows_v, acc, sem):
    cid = lax.axis_index("c")
    sid = lax.axis_index("s")
    wid = sid * NC + cid
    pltpu.sync_copy(zeros_hbm, acc.at[pl.ds(sid * RPT, RPT)])
    pltpu.sync_copy(srcp_hbm.at[wid], src_v)
    pltpu.sync_copy(dstp_hbm.at[wid], dst_v)
    plsc.subcore_barrier()

    def body(j, carry):
        pltpu.async_copy(g_hbm.at[src_v.at[j]], rows_v, sem).wait()
        pltpu.sync_copy(rows_v, acc.at[dst_v.at[j]], add=True)
        return carry

    lax.fori_loop(0, CH, body, 0)
    plsc.subcore_barrier()
    pltpu.sync_copy(acc.at[pl.ds(sid * RPT, RPT)],
                    out_hbm.at[cid, pl.ds(sid * RPT, RPT)])


def _mm_body(x_ref, w_ref, o_ref):
    o_ref[...] = jnp.dot(x_ref[...], w_ref[...],
                         preferred_element_type=jnp.float32)


def _tc_matmul(x, w):
    return pl.pallas_call(
        _mm_body,
        out_shape=jax.ShapeDtypeStruct((x.shape[0], w.shape[1]), jnp.float32),
    )(x, w)


def _scale_body(p_ref, dinv_ref, o_ref):
    o_ref[...] = p_ref[...] * dinv_ref[...]


def _tc_scale(p, dinv):
    return pl.pallas_call(
        _scale_body,
        out_shape=jax.ShapeDtypeStruct(p.shape, jnp.float32),
    )(p, dinv)


def _mid_body(s_ref, g_ref, dinv_ref, b_ref, w_ref, o_ref):
    s = s_ref[0, :N, :] + s_ref[1, :N, :] + g_ref[...]
    h = jnp.maximum(dinv_ref[...] * s + b_ref[...], 0.0)
    o_ref[...] = dinv_ref[...] * jnp.dot(h, w_ref[...],
                                         preferred_element_type=jnp.float32)


def _tc_mid(S, g, dinv, b, w):
    return pl.pallas_call(
        _mid_body,
        out_shape=jax.ShapeDtypeStruct((N, D), jnp.float32),
    )(S, g, dinv, b, w)


def _out_body(s_ref, g_ref, dinv_ref, b_ref, wfc_ref, bfc_ref, o_ref):
    s = s_ref[0, :N, :] + s_ref[1, :N, :] + g_ref[...]
    h = jnp.maximum(dinv_ref[...] * s + b_ref[...], 0.0)
    logit = jnp.dot(h, wfc_ref[...],
                    preferred_element_type=jnp.float32) + bfc_ref[...]
    col = lax.broadcasted_iota(jnp.int32, logit.shape, 1)
    valid = col < 2
    m = jnp.max(jnp.where(valid, logit, -1e30), axis=1, keepdims=True)
    ex = jnp.where(valid, jnp.exp(logit - m), 0.0)
    ssum = jnp.sum(ex, axis=1, keepdims=True)
    o_ref[...] = logit - m - jnp.log(ssum)


def _tc_out(S, g, dinv, b, wfc, bfc):
    return pl.pallas_call(
        _out_body,
        out_shape=jax.ShapeDtypeStruct((N, D), jnp.float32),
    )(S, g, dinv, b, wfc, bfc)


def kernel(x, edge_index, W1, b1, W2, b2, Wfc, bfc):
    src = edge_index[0]
    dst = edge_index[1]
    pad = EPAD - E
    ar = jnp.arange(pad, dtype=jnp.int32)
    srcp = jnp.concatenate([src, ar % N]).reshape(NW, CH, K)
    dstp = jnp.concatenate([dst, N + ar % (ACC_R - N)]).reshape(NW, CH, K)
    ones_k = jnp.ones((K,), jnp.float32)
    zeros1 = jnp.zeros((RPT,), jnp.float32)
    zeros2 = jnp.zeros((RPT, D), jnp.float32)

    degp = _sc_degree(dstp, ones_k, zeros1)
    p1 = _tc_matmul(x, W1)
    deg = degp[0, :N] + degp[1, :N] + 1.0
    dinv = lax.rsqrt(deg).reshape(N, 1)
    g1 = _tc_scale(p1, dinv)
    S1 = _sc_scatter(g1, srcp, dstp, zeros2)
    g2 = _tc_mid(S1, g1, dinv, b1.reshape(1, D), W2)
    S2 = _sc_scatter(g2, srcp, dstp, zeros2)
    wfc_pad = jnp.pad(Wfc, ((0, 0), (0, D - Wfc.shape[1])))
    bfc_pad = jnp.pad(bfc, (0, D - bfc.shape[0])).reshape(1, D)
    out = _tc_out(S2, g2, dinv, b2.reshape(1, D), wfc_pad, bfc_pad)
    return out[:, :2]


# speedup vs baseline: 31.9414x; 1.4128x over previous
import functools

import jax
import jax.numpy as jnp
from jax import lax
from jax.experimental import pallas as pl
from jax.experimental.pallas import tpu as pltpu
from jax.experimental.pallas import tpu_sc as plsc

N = 10000
D = 128
E = 320000
NC = 2
NS = 16
NW = NC * NS
K = 128
NB = 2
NI = 4
CH = 80
EPW = CH * K
EPAD = EPW * NW
ACC_R = 10240
RPT = ACC_R // NS

_mesh = plsc.VectorSubcoreMesh(core_axis_name="c", subcore_axis_name="s")


@functools.partial(
    pl.kernel,
    mesh=_mesh,
    out_type=jax.ShapeDtypeStruct((NC, ACC_R), jnp.float32),
    scratch_types=[
        pltpu.VMEM((CH, 2, K), jnp.int32),
        pltpu.VMEM((K,), jnp.float32),
        pltpu.VMEM_SHARED((ACC_R,), jnp.float32),
    ],
)
def _sc_degree(idxp_hbm, ones_hbm, zeros1_hbm, out_hbm, idx_v, ones_v, dacc):
    cid = lax.axis_index("c")
    sid = lax.axis_index("s")
    wid = sid * NC + cid
    pltpu.sync_copy(zeros1_hbm, dacc.at[pl.ds(sid * RPT, RPT)])
    pltpu.sync_copy(idxp_hbm.at[wid], idx_v)
    pltpu.sync_copy(ones_hbm, ones_v)
    plsc.subcore_barrier()

    def body(j, carry):
        pltpu.sync_copy(ones_v, dacc.at[idx_v.at[j, 1]], add=True)
        return carry

    lax.fori_loop(0, CH, body, 0)
    plsc.subcore_barrier()
    pltpu.sync_copy(dacc.at[pl.ds(sid * RPT, RPT)],
                    out_hbm.at[cid, pl.ds(sid * RPT, RPT)])


@functools.partial(
    pl.kernel,
    mesh=_mesh,
    out_type=jax.ShapeDtypeStruct((NC, ACC_R, D), jnp.float32),
    scratch_types=[
        pltpu.VMEM((NI, 2, K), jnp.int32),
        pltpu.VMEM((NB, K, D), jnp.float32),
        pltpu.VMEM_SHARED((ACC_R, D), jnp.float32),
    ] + [pltpu.SemaphoreType.DMA] * (NI + NB),
)
def _sc_scatter(g_hbm, idxp_hbm, zeros_hbm, out_hbm,
                idx_v, rows_v, acc, si0, si1, si2, si3, sg0, sg1):
    si = (si0, si1, si2, si3)
    sg = (sg0, sg1)
    cid = lax.axis_index("c")
    sid = lax.axis_index("s")
    wid = sid * NC + cid
    pltpu.sync_copy(zeros_hbm.at[pl.ds(sid * RPT, RPT)],
                    acc.at[pl.ds(sid * RPT, RPT)])
    plsc.subcore_barrier()

    for b in range(NI):
        pltpu.async_copy(idxp_hbm.at[wid, b], idx_v.at[b], si[b])

    def body(t, carry):
        j0 = NI * t
        for u in range(NI):
            j = j0 + u
            rb = u % NB
            pb = (u - 1) % NI
            pltpu.make_async_copy(idxp_hbm.at[wid, 0],
                                  idx_v.at[u], si[u]).wait()
            pltpu.async_copy(g_hbm.at[idx_v.at[u, 0]], rows_v.at[rb], sg[rb])

            def service_prev():
                pltpu.make_async_copy(g_hbm.at[idx_v.at[pb, 0]],
                                      rows_v.at[1 - rb], sg[1 - rb]).wait()
                pltpu.sync_copy(rows_v.at[1 - rb],
                                acc.at[idx_v.at[pb, 1]], add=True)

                @pl.when(j + NI - 1 < CH)
                def _():
                    pltpu.async_copy(idxp_hbm.at[wid, j + NI - 1],
                                     idx_v.at[pb], si[pb])

            if u == 0:
                pl.when(t > 0)(service_prev)
            else:
                service_prev()
        return carry

    lax.fori_loop(0, CH // NI, body, 0)
    pltpu.make_async_copy(g_hbm.at[idx_v.at[NI - 1, 0]],
                          rows_v.at[(CH - 1) % NB], sg[(CH - 1) % NB]).wait()
    pltpu.sync_copy(rows_v.at[(CH - 1) % NB],
                    acc.at[idx_v.at[NI - 1, 1]], add=True)
    plsc.subcore_barrier()
    pltpu.sync_copy(acc.at[pl.ds(sid * RPT, RPT)],
                    out_hbm.at[cid, pl.ds(sid * RPT, RPT)])


def _mm_body(x_ref, w_ref, o_ref):
    o_ref[...] = jnp.dot(x_ref[...], w_ref[...],
                         preferred_element_type=jnp.float32)


def _tc_matmul(x, w):
    return pl.pallas_call(
        _mm_body,
        out_shape=jax.ShapeDtypeStruct((x.shape[0], w.shape[1]), jnp.float32),
    )(x, w)


def _scale_body(p_ref, dinv_ref, o_ref):
    o_ref[...] = p_ref[...] * dinv_ref[...]


def _tc_scale(p, dinv):
    return pl.pallas_call(
        _scale_body,
        out_shape=jax.ShapeDtypeStruct(p.shape, jnp.float32),
    )(p, dinv)


def _mid_body(s_ref, g_ref, dinv_ref, b_ref, w_ref, o_ref):
    s = s_ref[0, :N, :] + s_ref[1, :N, :] + g_ref[...]
    h = jnp.maximum(dinv_ref[...] * s + b_ref[...], 0.0)
    o_ref[...] = dinv_ref[...] * jnp.dot(h, w_ref[...],
                                         preferred_element_type=jnp.float32)


def _tc_mid(S, g, dinv, b, w):
    return pl.pallas_call(
        _mid_body,
        out_shape=jax.ShapeDtypeStruct((N, D), jnp.float32),
    )(S, g, dinv, b, w)


def _out_body(s_ref, g_ref, dinv_ref, b_ref, wfc_ref, bfc_ref, o_ref):
    s = s_ref[0, :N, :] + s_ref[1, :N, :] + g_ref[...]
    h = jnp.maximum(dinv_ref[...] * s + b_ref[...], 0.0)
    logit = jnp.dot(h, wfc_ref[...],
                    preferred_element_type=jnp.float32) + bfc_ref[...]
    col = lax.broadcasted_iota(jnp.int32, logit.shape, 1)
    valid = col < 2
    m = jnp.max(jnp.where(valid, logit, -1e30), axis=1, keepdims=True)
    ex = jnp.where(valid, jnp.exp(logit - m), 0.0)
    ssum = jnp.sum(ex, axis=1, keepdims=True)
    o_ref[...] = logit - m - jnp.log(ssum)


def _tc_out(S, g, dinv, b, wfc, bfc):
    return pl.pallas_call(
        _out_body,
        out_shape=jax.ShapeDtypeStruct((N, D), jnp.float32),
    )(S, g, dinv, b, wfc, bfc)


def kernel(x, edge_index, W1, b1, W2, b2, Wfc, bfc):
    src = edge_index[0]
    dst = edge_index[1]
    pad = EPAD - E
    ar = jnp.arange(pad, dtype=jnp.int32)
    srcp = jnp.concatenate([src, ar % N]).reshape(NW, CH, 1, K)
    dstp = jnp.concatenate([dst, N + ar % (ACC_R - N)]).reshape(NW, CH, 1, K)
    idxp = jnp.concatenate([srcp, dstp], axis=2)
    ones_k = jnp.ones((K,), jnp.float32)
    zeros1 = jnp.zeros((RPT,), jnp.float32)
    zeros2 = jnp.zeros((ACC_R, D), jnp.float32)

    degp = _sc_degree(idxp, ones_k, zeros1)
    p1 = _tc_matmul(x, W1)
    deg = degp[0, :N] + degp[1, :N] + 1.0
    dinv = lax.rsqrt(deg).reshape(N, 1)
    g1 = _tc_scale(p1, dinv)
    S1 = _sc_scatter(g1, idxp, zeros2)
    g2 = _tc_mid(S1, g1, dinv, b1.reshape(1, D), W2)
    S2 = _sc_scatter(g2, idxp, zeros2)
    wfc_pad = jnp.pad(Wfc, ((0, 0), (0, D - Wfc.shape[1])))
    bfc_pad = jnp.pad(bfc, (0, D - bfc.shape[0])).reshape(1, D)
    out = _tc_out(S2, g2, dinv, b2.reshape(1, D), wfc_pad, bfc_pad)
    return out[:, :2]


# speedup vs baseline: 33.1457x; 1.0377x over previous
import functools

import jax
import jax.numpy as jnp
from jax import lax
from jax.experimental import pallas as pl
from jax.experimental.pallas import tpu as pltpu
from jax.experimental.pallas import tpu_sc as plsc

N = 10000
D = 128
E = 320000
NC = 2
NS = 16
NW = NC * NS
K = 128
NB = 2
NI = 4
CH = 80
EPW = CH * K
EPAD = EPW * NW
ACC_R = 10240
RPT = ACC_R // NS
G_TAIL = N - (NS - 1) * RPT

_mesh = plsc.VectorSubcoreMesh(core_axis_name="c", subcore_axis_name="s")


@functools.partial(
    pl.kernel,
    mesh=_mesh,
    out_type=jax.ShapeDtypeStruct((NC, ACC_R), jnp.float32),
    scratch_types=[
        pltpu.VMEM((CH, 2, K), jnp.int32),
        pltpu.VMEM((K,), jnp.float32),
        pltpu.VMEM((640,), jnp.float32),
        pltpu.VMEM_SHARED((ACC_R,), jnp.float32),
    ],
)
def _sc_degree(idxp_hbm, out_hbm, idx_v, ones_v, init_v, dacc):
    cid = lax.axis_index("c")
    sid = lax.axis_index("s")
    wid = sid * NC + cid
    pltpu.sync_copy(idxp_hbm.at[wid], idx_v)
    one = jnp.float32(1.0)
    for i in range(K // 16):
        ones_v[pl.ds(16 * i, 16)] = jnp.full((16,), one, jnp.float32)
    init = jnp.where(cid == 0, one, jnp.float32(0.0))
    for i in range(640 // 16):
        init_v[pl.ds(16 * i, 16)] = jnp.full((16,), init, jnp.float32)
    pltpu.sync_copy(init_v.at[pl.ds(0, RPT)], dacc.at[pl.ds(sid * RPT, RPT)])
    plsc.subcore_barrier()

    def body(j, carry):
        pltpu.sync_copy(ones_v, dacc.at[idx_v.at[j, 1]], add=True)
        return carry

    lax.fori_loop(0, CH, body, 0)
    plsc.subcore_barrier()
    pltpu.sync_copy(dacc.at[pl.ds(sid * RPT, RPT)],
                    out_hbm.at[cid, pl.ds(sid * RPT, RPT)])


@functools.partial(
    pl.kernel,
    mesh=_mesh,
    out_type=jax.ShapeDtypeStruct((NC, ACC_R, D), jnp.float32),
    scratch_types=[
        pltpu.VMEM((NI, 2, K), jnp.int32),
        pltpu.VMEM((NB, K, D), jnp.float32),
        pltpu.VMEM_SHARED((ACC_R, D), jnp.float32),
    ] + [pltpu.SemaphoreType.DMA] * (NI + NB),
)
def _sc_scatter(g_hbm, idxp_hbm, zeros_hbm, out_hbm,
                idx_v, rows_v, acc, si0, si1, si2, si3, sg0, sg1):
    si = (si0, si1, si2, si3)
    sg = (sg0, sg1)
    cid = lax.axis_index("c")
    sid = lax.axis_index("s")
    wid = sid * NC + cid

    @pl.when(cid == 0)
    def _():
        @pl.when(sid < NS - 1)
        def _():
            pltpu.sync_copy(g_hbm.at[pl.ds(sid * RPT, RPT)],
                            acc.at[pl.ds(sid * RPT, RPT)])

        @pl.when(sid == NS - 1)
        def _():
            pltpu.sync_copy(g_hbm.at[pl.ds((NS - 1) * RPT, G_TAIL)],
                            acc.at[pl.ds((NS - 1) * RPT, G_TAIL)])
            pltpu.sync_copy(g_hbm.at[pl.ds(0, ACC_R - N)],
                            acc.at[pl.ds(N, ACC_R - N)])

    @pl.when(cid == 1)
    def _():
        pltpu.sync_copy(zeros_hbm.at[pl.ds(sid * RPT, RPT)],
                        acc.at[pl.ds(sid * RPT, RPT)])

    plsc.subcore_barrier()

    for b in range(NI):
        pltpu.async_copy(idxp_hbm.at[wid, b], idx_v.at[b], si[b])

    def body(t, carry):
        j0 = NI * t
        for u in range(NI):
            j = j0 + u
            rb = u % NB
            pb = (u - 1) % NI
            pltpu.make_async_copy(idxp_hbm.at[wid, 0],
                                  idx_v.at[u], si[u]).wait()
            pltpu.async_copy(g_hbm.at[idx_v.at[u, 0]], rows_v.at[rb], sg[rb])

            def service_prev():
                pltpu.make_async_copy(g_hbm.at[idx_v.at[pb, 0]],
                                      rows_v.at[1 - rb], sg[1 - rb]).wait()
                pltpu.sync_copy(rows_v.at[1 - rb],
                                acc.at[idx_v.at[pb, 1]], add=True)

                @pl.when(j + NI - 1 < CH)
                def _():
                    pltpu.async_copy(idxp_hbm.at[wid, j + NI - 1],
                                     idx_v.at[pb], si[pb])

            if u == 0:
                pl.when(t > 0)(service_prev)
            else:
                service_prev()
        return carry

    lax.fori_loop(0, CH // NI, body, 0)
    pltpu.make_async_copy(g_hbm.at[idx_v.at[NI - 1, 0]],
                          rows_v.at[(CH - 1) % NB], sg[(CH - 1) % NB]).wait()
    pltpu.sync_copy(rows_v.at[(CH - 1) % NB],
                    acc.at[idx_v.at[NI - 1, 1]], add=True)
    plsc.subcore_barrier()
    pltpu.sync_copy(acc.at[pl.ds(sid * RPT, RPT)],
                    out_hbm.at[cid, pl.ds(sid * RPT, RPT)])


def _mm_body(x_ref, w_ref, dinv_ref, o_ref):
    o_ref[...] = dinv_ref[...] * jnp.dot(x_ref[...], w_ref[...],
                                         preferred_element_type=jnp.float32)


def _tc_mm_scale(x, w, dinv):
    return pl.pallas_call(
        _mm_body,
        out_shape=jax.ShapeDtypeStruct((x.shape[0], w.shape[1]), jnp.float32),
    )(x, w, dinv)


def _mid_body(s_ref, dinv_ref, b_ref, w_ref, o_ref):
    s = s_ref[0, :N, :] + s_ref[1, :N, :]
    h = jnp.maximum(dinv_ref[...] * s + b_ref[...], 0.0)
    o_ref[...] = dinv_ref[...] * jnp.dot(h, w_ref[...],
                                         preferred_element_type=jnp.float32)


def _tc_mid(S, dinv, b, w):
    return pl.pallas_call(
        _mid_body,
        out_shape=jax.ShapeDtypeStruct((N, D), jnp.float32),
    )(S, dinv, b, w)


def _out_body(s_ref, dinv_ref, b_ref, wfc_ref, bfc_ref, o_ref):
    s = s_ref[0, :N, :] + s_ref[1, :N, :]
    h = jnp.maximum(dinv_ref[...] * s + b_ref[...], 0.0)
    logit = jnp.dot(h, wfc_ref[...],
                    preferred_element_type=jnp.float32) + bfc_ref[...]
    col = lax.broadcasted_iota(jnp.int32, logit.shape, 1)
    valid = col < 2
    m = jnp.max(jnp.where(valid, logit, -1e30), axis=1, keepdims=True)
    ex = jnp.where(valid, jnp.exp(logit - m), 0.0)
    ssum = jnp.sum(ex, axis=1, keepdims=True)
    o_ref[...] = logit - m - jnp.log(ssum)


def _tc_out(S, dinv, b, wfc, bfc):
    return pl.pallas_call(
        _out_body,
        out_shape=jax.ShapeDtypeStruct((N, D), jnp.float32),
    )(S, dinv, b, wfc, bfc)


def kernel(x, edge_index, W1, b1, W2, b2, Wfc, bfc):
    src = edge_index[0]
    dst = edge_index[1]
    pad = EPAD - E
    ar = jnp.arange(pad, dtype=jnp.int32)
    srcp = jnp.concatenate([src, ar % N]).reshape(NW, CH, 1, K)
    dstp = jnp.concatenate([dst, N + ar % (ACC_R - N)]).reshape(NW, CH, 1, K)
    idxp = jnp.concatenate([srcp, dstp], axis=2)
    zeros2 = jnp.zeros((ACC_R, D), jnp.float32)

    degp = _sc_degree(idxp)
    deg = degp[0, :N] + degp[1, :N]
    dinv = lax.rsqrt(deg).reshape(N, 1)
    g1 = _tc_mm_scale(x, W1, dinv)
    S1 = _sc_scatter(g1, idxp, zeros2)
    g2 = _tc_mid(S1, dinv, b1.reshape(1, D), W2)
    S2 = _sc_scatter(g2, idxp, zeros2)
    wfc_pad = jnp.pad(Wfc, ((0, 0), (0, D - Wfc.shape[1])))
    bfc_pad = jnp.pad(bfc, (0, D - bfc.shape[0])).reshape(1, D)
    out = _tc_out(S2, dinv, b2.reshape(1, D), wfc_pad, bfc_pad)
    return out[:, :2]


# speedup vs baseline: 33.3297x; 1.0056x over previous
import functools

import jax
import jax.numpy as jnp
from jax import lax
from jax.experimental import pallas as pl
from jax.experimental.pallas import tpu as pltpu
from jax.experimental.pallas import tpu_sc as plsc

N = 10000
D = 128
E = 320000
NC = 2
NS = 16
NW = NC * NS
K = 128
NB = 2
NI = 4
CH = 80
EPW = CH * K
EPAD = EPW * NW
ACC_R = 10240
RPT = ACC_R // NS
G_TAIL = N - (NS - 1) * RPT

_mesh = plsc.VectorSubcoreMesh(core_axis_name="c", subcore_axis_name="s")


@functools.partial(
    pl.kernel,
    mesh=_mesh,
    out_type=jax.ShapeDtypeStruct((NC, ACC_R), jnp.float32),
    scratch_types=[
        pltpu.VMEM((CH, 2, K), jnp.int32),
        pltpu.VMEM((K,), jnp.float32),
        pltpu.VMEM((640,), jnp.float32),
        pltpu.VMEM_SHARED((ACC_R,), jnp.float32),
    ],
)
def _sc_degree(idxp_hbm, out_hbm, idx_v, ones_v, init_v, dacc):
    cid = lax.axis_index("c")
    sid = lax.axis_index("s")
    wid = sid * NC + cid
    pltpu.sync_copy(idxp_hbm.at[wid], idx_v)
    one = jnp.float32(1.0)
    for i in range(K // 16):
        ones_v[pl.ds(16 * i, 16)] = jnp.full((16,), one, jnp.float32)
    init = jnp.where(cid == 0, one, jnp.float32(0.0))
    for i in range(640 // 16):
        init_v[pl.ds(16 * i, 16)] = jnp.full((16,), init, jnp.float32)
    pltpu.sync_copy(init_v.at[pl.ds(0, RPT)], dacc.at[pl.ds(sid * RPT, RPT)])
    plsc.subcore_barrier()

    def body(j, carry):
        pltpu.sync_copy(ones_v, dacc.at[idx_v.at[j, 1]], add=True)
        return carry

    lax.fori_loop(0, CH, body, 0)
    plsc.subcore_barrier()
    pltpu.sync_copy(dacc.at[pl.ds(sid * RPT, RPT)],
                    out_hbm.at[cid, pl.ds(sid * RPT, RPT)])


@functools.partial(
    pl.kernel,
    mesh=_mesh,
    out_type=jax.ShapeDtypeStruct((NC, ACC_R, D), jnp.float32),
    scratch_types=[
        pltpu.VMEM((NI, 2, K), jnp.int32),
        pltpu.VMEM((NB, K, D), jnp.float32),
        pltpu.VMEM_SHARED((ACC_R, D), jnp.float32),
    ] + [pltpu.SemaphoreType.DMA] * (NI + 2 * NB),
)
def _sc_scatter(g_hbm, idxp_hbm, zeros_hbm, out_hbm,
                idx_v, rows_v, acc, si0, si1, si2, si3, sg0, sg1, ss0, ss1):
    si = (si0, si1, si2, si3)
    sg = (sg0, sg1)
    ss = (ss0, ss1)
    cid = lax.axis_index("c")
    sid = lax.axis_index("s")
    wid = sid * NC + cid

    @pl.when(cid == 0)
    def _():
        @pl.when(sid < NS - 1)
        def _():
            pltpu.sync_copy(g_hbm.at[pl.ds(sid * RPT, RPT)],
                            acc.at[pl.ds(sid * RPT, RPT)])

        @pl.when(sid == NS - 1)
        def _():
            pltpu.sync_copy(g_hbm.at[pl.ds((NS - 1) * RPT, G_TAIL)],
                            acc.at[pl.ds((NS - 1) * RPT, G_TAIL)])
            pltpu.sync_copy(g_hbm.at[pl.ds(0, ACC_R - N)],
                            acc.at[pl.ds(N, ACC_R - N)])

    @pl.when(cid == 1)
    def _():
        pltpu.sync_copy(zeros_hbm.at[pl.ds(sid * RPT, RPT)],
                        acc.at[pl.ds(sid * RPT, RPT)])

    plsc.subcore_barrier()

    for b in range(NI):
        pltpu.async_copy(idxp_hbm.at[wid, b], idx_v.at[b], si[b])

    def body(t, carry):
        j0 = NI * t
        for u in range(NI):
            j = j0 + u
            rb = u % NB
            pb = (u - 1) % NI
            qb = (u - 2) % NI
            pltpu.make_async_copy(idxp_hbm.at[wid, 0],
                                  idx_v.at[u], si[u]).wait()

            def wait_prev_scatter():
                pltpu.make_async_copy(rows_v.at[rb],
                                      acc.at[idx_v.at[qb, 1]], ss[rb]).wait()

            if u < 2:
                pl.when(t > 0)(wait_prev_scatter)
            else:
                wait_prev_scatter()
            pltpu.async_copy(g_hbm.at[idx_v.at[u, 0]], rows_v.at[rb], sg[rb])

            def service_prev():
                pltpu.make_async_copy(g_hbm.at[idx_v.at[pb, 0]],
                                      rows_v.at[1 - rb], sg[1 - rb]).wait()
                pltpu.async_copy(rows_v.at[1 - rb],
                                 acc.at[idx_v.at[pb, 1]], ss[1 - rb], add=True)

            if u == 0:
                pl.when(t > 0)(service_prev)
            else:
                service_prev()

            @pl.when(jnp.logical_and(j >= 2, j + 2 < CH))
            def _():
                pltpu.async_copy(idxp_hbm.at[wid, j + 2], idx_v.at[qb], si[qb])
        return carry

    lax.fori_loop(0, CH // NI, body, 0)
    lb = (CH - 1) % NB
    pltpu.make_async_copy(g_hbm.at[idx_v.at[NI - 1, 0]],
                          rows_v.at[lb], sg[lb]).wait()
    pltpu.async_copy(rows_v.at[lb], acc.at[idx_v.at[NI - 1, 1]], ss[lb],
                     add=True)
    pltpu.make_async_copy(rows_v.at[1 - lb],
                          acc.at[idx_v.at[NI - 2, 1]], ss[1 - lb]).wait()
    pltpu.make_async_copy(rows_v.at[lb],
                          acc.at[idx_v.at[NI - 1, 1]], ss[lb]).wait()
    plsc.subcore_barrier()
    pltpu.sync_copy(acc.at[pl.ds(sid * RPT, RPT)],
                    out_hbm.at[cid, pl.ds(sid * RPT, RPT)])


def _mm_body(x_ref, w_ref, dinv_ref, o_ref):
    o_ref[...] = dinv_ref[...] * jnp.dot(x_ref[...], w_ref[...],
                                         preferred_element_type=jnp.float32)


def _tc_mm_scale(x, w, dinv):
    return pl.pallas_call(
        _mm_body,
        out_shape=jax.ShapeDtypeStruct((x.shape[0], w.shape[1]), jnp.float32),
    )(x, w, dinv)


def _mid_body(s_ref, dinv_ref, b_ref, w_ref, o_ref):
    s = s_ref[0, :N, :] + s_ref[1, :N, :]
    h = jnp.maximum(dinv_ref[...] * s + b_ref[...], 0.0)
    o_ref[...] = dinv_ref[...] * jnp.dot(h, w_ref[...],
                                         preferred_element_type=jnp.float32)


def _tc_mid(S, dinv, b, w):
    return pl.pallas_call(
        _mid_body,
        out_shape=jax.ShapeDtypeStruct((N, D), jnp.float32),
    )(S, dinv, b, w)


def _out_body(s_ref, dinv_ref, b_ref, wfc_ref, bfc_ref, o_ref):
    s = s_ref[0, :N, :] + s_ref[1, :N, :]
    h = jnp.maximum(dinv_ref[...] * s + b_ref[...], 0.0)
    logit = jnp.dot(h, wfc_ref[...],
                    preferred_element_type=jnp.float32) + bfc_ref[...]
    col = lax.broadcasted_iota(jnp.int32, logit.shape, 1)
    valid = col < 2
    m = jnp.max(jnp.where(valid, logit, -1e30), axis=1, keepdims=True)
    ex = jnp.where(valid, jnp.exp(logit - m), 0.0)
    ssum = jnp.sum(ex, axis=1, keepdims=True)
    o_ref[...] = logit - m - jnp.log(ssum)


def _tc_out(S, dinv, b, wfc, bfc):
    return pl.pallas_call(
        _out_body,
        out_shape=jax.ShapeDtypeStruct((N, D), jnp.float32),
    )(S, dinv, b, wfc, bfc)


def kernel(x, edge_index, W1, b1, W2, b2, Wfc, bfc):
    src = edge_index[0]
    dst = edge_index[1]
    pad = EPAD - E
    ar = jnp.arange(pad, dtype=jnp.int32)
    srcp = jnp.concatenate([src, ar % N]).reshape(NW, CH, 1, K)
    dstp = jnp.concatenate([dst, N + ar % (ACC_R - N)]).reshape(NW, CH, 1, K)
    idxp = jnp.concatenate([srcp, dstp], axis=2)
    zeros2 = jnp.zeros((ACC_R, D), jnp.float32)

    degp = _sc_degree(idxp)
    deg = degp[0, :N] + degp[1, :N]
    dinv = lax.rsqrt(deg).reshape(N, 1)
    g1 = _tc_mm_scale(x, W1, dinv)
    S1 = _sc_scatter(g1, idxp, zeros2)
    g2 = _tc_mid(S1, dinv, b1.reshape(1, D), W2)
    S2 = _sc_scatter(g2, idxp, zeros2)
    wfc_pad = jnp.pad(Wfc, ((0, 0), (0, D - Wfc.shape[1])))
    bfc_pad = jnp.pad(bfc, (0, D - bfc.shape[0])).reshape(1, D)
    out = _tc_out(S2, dinv, b2.reshape(1, D), wfc_pad, bfc_pad)
    return out[:, :2]


# speedup vs baseline: 34.0721x; 1.0223x over previous
import functools

import numpy as np

import jax
import jax.numpy as jnp
from jax import lax
from jax.experimental import pallas as pl
from jax.experimental.pallas import tpu as pltpu
from jax.experimental.pallas import tpu_sc as plsc

N = 10000
D = 128
E = 320000
NC = 2
NS = 16
NW = NC * NS
EPW = E // NW
K = 128
CHF = EPW // K
TAIL = EPW - CHF * K
NB = 2
NI = 6
ACC_R = 10240
RPT = ACC_R // NS
G_TAIL = N - (NS - 1) * RPT

_mesh = plsc.VectorSubcoreMesh(core_axis_name="c", subcore_axis_name="s")

_ZEROS = np.zeros((ACC_R, D), np.float32)


@functools.partial(
    pl.kernel,
    mesh=_mesh,
    out_type=jax.ShapeDtypeStruct((NC, ACC_R), jnp.float32),
    scratch_types=[
        pltpu.VMEM((3, K), jnp.int32),
        pltpu.VMEM((16,), jnp.int32),
        pltpu.VMEM((K,), jnp.float32),
        pltpu.VMEM((640,), jnp.float32),
        pltpu.VMEM_SHARED((ACC_R,), jnp.float32),
    ] + [pltpu.SemaphoreType.DMA] * 3,
)
def _sc_degree(ei_hbm, out_hbm, idx_v, tidx_v, ones_v, init_v, dacc,
               si0, si1, si2):
    si = (si0, si1, si2)
    cid = lax.axis_index("c")
    sid = lax.axis_index("s")
    wid = sid * NC + cid
    for b in range(3):
        pltpu.async_copy(ei_hbm.at[1, wid, pl.ds(b * K, K)],
                         idx_v.at[b], si[b])
    one = jnp.float32(1.0)
    for i in range(K // 16):
        ones_v[pl.ds(16 * i, 16)] = jnp.full((16,), one, jnp.float32)
    init = jnp.where(cid == 0, one, jnp.float32(0.0))
    for i in range(640 // 16):
        init_v[pl.ds(16 * i, 16)] = jnp.full((16,), init, jnp.float32)
    pltpu.sync_copy(init_v.at[pl.ds(0, RPT)], dacc.at[pl.ds(sid * RPT, RPT)])
    plsc.subcore_barrier()

    def body(t, carry):
        j0 = 3 * t
        for u in range(3):
            j = j0 + u
            pltpu.make_async_copy(ei_hbm.at[1, wid, pl.ds(0, K)],
                                  idx_v.at[u], si[u]).wait()
            pltpu.sync_copy(ones_v, dacc.at[idx_v.at[u]], add=True)

            @pl.when(j + 3 < CHF)
            def _():
                pltpu.async_copy(ei_hbm.at[1, wid, pl.ds((j + 3) * K, K)],
                                 idx_v.at[u], si[u])
        return carry

    lax.fori_loop(0, CHF // 3, body, 0)
    pltpu.sync_copy(ei_hbm.at[1, wid, pl.ds(CHF * K, TAIL)], tidx_v)
    pltpu.sync_copy(ones_v.at[pl.ds(0, TAIL)], dacc.at[tidx_v], add=True)
    plsc.subcore_barrier()
    pltpu.sync_copy(dacc.at[pl.ds(sid * RPT, RPT)],
                    out_hbm.at[cid, pl.ds(sid * RPT, RPT)])


@functools.partial(
    pl.kernel,
    mesh=_mesh,
    out_type=jax.ShapeDtypeStruct((NC, ACC_R, D), jnp.float32),
    scratch_types=[
        pltpu.VMEM((NI, K), jnp.int32),
        pltpu.VMEM((NI, K), jnp.int32),
        pltpu.VMEM((16,), jnp.int32),
        pltpu.VMEM((16,), jnp.int32),
        pltpu.VMEM((NB, K, D), jnp.float32),
        pltpu.VMEM_SHARED((ACC_R, D), jnp.float32),
    ] + [pltpu.SemaphoreType.DMA] * (2 * NI + 2 * NB),
)
def _sc_scatter(g_hbm, ei_hbm, zeros_hbm, out_hbm,
                sidx_v, didx_v, tsrc_v, tdst_v, rows_v, acc,
                sa0, sa1, sa2, sa3, sa4, sa5,
                sb0, sb1, sb2, sb3, sb4, sb5,
                sg0, sg1, ss0, ss1):
    sa = (sa0, sa1, sa2, sa3, sa4, sa5)
    sb = (sb0, sb1, sb2, sb3, sb4, sb5)
    sg = (sg0, sg1)
    ss = (ss0, ss1)
    cid = lax.axis_index("c")
    sid = lax.axis_index("s")
    wid = sid * NC + cid

    for b in range(NI):
        pltpu.async_copy(ei_hbm.at[0, wid, pl.ds(b * K, K)],
                         sidx_v.at[b], sa[b])
        pltpu.async_copy(ei_hbm.at[1, wid, pl.ds(b * K, K)],
                         didx_v.at[b], sb[b])

    @pl.when(cid == 0)
    def _():
        @pl.when(sid < NS - 1)
        def _():
            pltpu.sync_copy(g_hbm.at[pl.ds(sid * RPT, RPT)],
                            acc.at[pl.ds(sid * RPT, RPT)])

        @pl.when(sid == NS - 1)
        def _():
            pltpu.sync_copy(g_hbm.at[pl.ds((NS - 1) * RPT, G_TAIL)],
                            acc.at[pl.ds((NS - 1) * RPT, G_TAIL)])
            pltpu.sync_copy(g_hbm.at[pl.ds(0, ACC_R - N)],
                            acc.at[pl.ds(N, ACC_R - N)])

    @pl.when(cid == 1)
    def _():
        pltpu.sync_copy(zeros_hbm.at[pl.ds(sid * RPT, RPT)],
                        acc.at[pl.ds(sid * RPT, RPT)])

    plsc.subcore_barrier()

    def body(t, carry):
        j0 = NI * t
        for u in range(NI):
            j = j0 + u
            rb = u % NB
            pb = (u - 1) % NI
            qb = (u - 2) % NI
            pltpu.make_async_copy(ei_hbm.at[0, wid, pl.ds(0, K)],
                                  sidx_v.at[u], sa[u]).wait()
            pltpu.make_async_copy(ei_hbm.at[1, wid, pl.ds(0, K)],
                                  didx_v.at[u], sb[u]).wait()

            def wait_prev_scatter():
                pltpu.make_async_copy(rows_v.at[rb],
                                      acc.at[didx_v.at[qb]], ss[rb]).wait()

            if u < 2:
                pl.when(t > 0)(wait_prev_scatter)
            else:
                wait_prev_scatter()
            pltpu.async_copy(g_hbm.at[sidx_v.at[u]], rows_v.at[rb], sg[rb])

            def service_prev():
                pltpu.make_async_copy(g_hbm.at[sidx_v.at[pb]],
                                      rows_v.at[1 - rb], sg[1 - rb]).wait()
                pltpu.async_copy(rows_v.at[1 - rb],
                                 acc.at[didx_v.at[pb]], ss[1 - rb], add=True)

            if u == 0:
                pl.when(t > 0)(service_prev)
            else:
                service_prev()

            @pl.when(jnp.logical_and(j >= 2, j + NI - 2 < CHF))
            def _():
                pltpu.async_copy(ei_hbm.at[0, wid, pl.ds((j + NI - 2) * K, K)],
                                 sidx_v.at[qb], sa[qb])
                pltpu.async_copy(ei_hbm.at[1, wid, pl.ds((j + NI - 2) * K, K)],
                                 didx_v.at[qb], sb[qb])
        return carry

    lax.fori_loop(0, CHF // NI, body, 0)
    lb = (CHF - 1) % NB
    pltpu.make_async_copy(g_hbm.at[sidx_v.at[NI - 1]],
                          rows_v.at[lb], sg[lb]).wait()
    pltpu.async_copy(rows_v.at[lb], acc.at[didx_v.at[NI - 1]], ss[lb],
                     add=True)
    pltpu.make_async_copy(rows_v.at[1 - lb],
                          acc.at[didx_v.at[NI - 2]], ss[1 - lb]).wait()
    pltpu.make_async_copy(rows_v.at[lb],
                          acc.at[didx_v.at[NI - 1]], ss[lb]).wait()
    pltpu.sync_copy(ei_hbm.at[0, wid, pl.ds(CHF * K, TAIL)], tsrc_v)
    pltpu.sync_copy(ei_hbm.at[1, wid, pl.ds(CHF * K, TAIL)], tdst_v)
    pltpu.async_copy(g_hbm.at[tsrc_v], rows_v.at[0, pl.ds(0, TAIL)],
                     sg[0]).wait()
    pltpu.sync_copy(rows_v.at[0, pl.ds(0, TAIL)], acc.at[tdst_v], add=True)
    plsc.subcore_barrier()
    pltpu.sync_copy(acc.at[pl.ds(sid * RPT, RPT)],
                    out_hbm.at[cid, pl.ds(sid * RPT, RPT)])


def _mm_body(x_ref, w_ref, dinv_ref, o_ref):
    o_ref[...] = dinv_ref[...] * jnp.dot(x_ref[...], w_ref[...],
                                         preferred_element_type=jnp.float32)


def _tc_mm_scale(x, w, dinv):
    return pl.pallas_call(
        _mm_body,
        out_shape=jax.ShapeDtypeStruct((x.shape[0], w.shape[1]), jnp.float32),
    )(x, w, dinv)


def _mid_body(s_ref, dinv_ref, b_ref, w_ref, o_ref):
    s = s_ref[0, :N, :] + s_ref[1, :N, :]
    h = jnp.maximum(dinv_ref[...] * s + b_ref[...], 0.0)
    o_ref[...] = dinv_ref[...] * jnp.dot(h, w_ref[...],
                                         preferred_element_type=jnp.float32)


def _tc_mid(S, dinv, b, w):
    return pl.pallas_call(
        _mid_body,
        out_shape=jax.ShapeDtypeStruct((N, D), jnp.float32),
    )(S, dinv, b, w)


def _out_body(s_ref, dinv_ref, b_ref, wfc_ref, bfc_ref, o_ref):
    s = s_ref[0, :N, :] + s_ref[1, :N, :]
    h = jnp.maximum(dinv_ref[...] * s + b_ref[...], 0.0)
    logit = jnp.dot(h, wfc_ref[...],
                    preferred_element_type=jnp.float32) + bfc_ref[...]
    m = jnp.max(logit, axis=1, keepdims=True)
    ssum = jnp.sum(jnp.exp(logit - m), axis=1, keepdims=True)
    o_ref[...] = logit - m - jnp.log(ssum)


def _tc_out(S, dinv, b, wfc, bfc):
    return pl.pallas_call(
        _out_body,
        out_shape=jax.ShapeDtypeStruct((N, 2), jnp.float32),
    )(S, dinv, b, wfc, bfc)


def kernel(x, edge_index, W1, b1, W2, b2, Wfc, bfc):
    ei = edge_index.reshape(2, NW, EPW)
    zeros2 = jnp.asarray(_ZEROS)

    degp = _sc_degree(ei)
    deg = degp[0, :N] + degp[1, :N]
    dinv = lax.rsqrt(deg).reshape(N, 1)
    g1 = _tc_mm_scale(x, W1, dinv)
    S1 = _sc_scatter(g1, ei, zeros2)
    g2 = _tc_mid(S1, dinv, b1.reshape(1, D), W2)
    S2 = _sc_scatter(g2, ei, zeros2)
    return _tc_out(S2, dinv, b2.reshape(1, D), Wfc, bfc.reshape(1, 2))


# speedup vs baseline: 34.4463x; 1.0110x over previous
import functools

import jax
import jax.numpy as jnp
from jax import lax
from jax.experimental import pallas as pl
from jax.experimental.pallas import tpu as pltpu
from jax.experimental.pallas import tpu_sc as plsc

N = 10000
D = 128
E = 320000
NC = 2
NS = 16
NW = NC * NS
EPW = E // NW
K = 128
CHF = EPW // K
TAIL = EPW - CHF * K
NB = 2
NI = 6
ACC_R = 10240
RPT = ACC_R // NS
G_TAIL = N - (NS - 1) * RPT

_mesh = plsc.VectorSubcoreMesh(core_axis_name="c", subcore_axis_name="s")


@functools.partial(
    pl.kernel,
    mesh=_mesh,
    out_type=jax.ShapeDtypeStruct((NC, ACC_R), jnp.float32),
    scratch_types=[
        pltpu.VMEM((3, K), jnp.int32),
        pltpu.VMEM((16,), jnp.int32),
        pltpu.VMEM((K,), jnp.float32),
        pltpu.VMEM((640,), jnp.float32),
        pltpu.VMEM_SHARED((ACC_R,), jnp.float32),
    ] + [pltpu.SemaphoreType.DMA] * 3,
)
def _sc_degree(ei_hbm, out_hbm, idx_v, tidx_v, ones_v, init_v, dacc,
               si0, si1, si2):
    si = (si0, si1, si2)
    cid = lax.axis_index("c")
    sid = lax.axis_index("s")
    wid = sid * NC + cid
    for b in range(3):
        pltpu.async_copy(ei_hbm.at[1, wid, pl.ds(b * K, K)],
                         idx_v.at[b], si[b])
    one = jnp.float32(1.0)
    for i in range(K // 16):
        ones_v[pl.ds(16 * i, 16)] = jnp.full((16,), one, jnp.float32)
    init = jnp.where(cid == 0, one, jnp.float32(0.0))
    for i in range(640 // 16):
        init_v[pl.ds(16 * i, 16)] = jnp.full((16,), init, jnp.float32)
    pltpu.sync_copy(init_v.at[pl.ds(0, RPT)], dacc.at[pl.ds(sid * RPT, RPT)])
    plsc.subcore_barrier()

    def body(t, carry):
        j0 = 3 * t
        for u in range(3):
            j = j0 + u
            pltpu.make_async_copy(ei_hbm.at[1, wid, pl.ds(0, K)],
                                  idx_v.at[u], si[u]).wait()
            pltpu.sync_copy(ones_v, dacc.at[idx_v.at[u]], add=True)

            @pl.when(j + 3 < CHF)
            def _():
                pltpu.async_copy(ei_hbm.at[1, wid, pl.ds((j + 3) * K, K)],
                                 idx_v.at[u], si[u])
        return carry

    lax.fori_loop(0, CHF // 3, body, 0)
    pltpu.sync_copy(ei_hbm.at[1, wid, pl.ds(CHF * K, TAIL)], tidx_v)
    pltpu.sync_copy(ones_v.at[pl.ds(0, TAIL)], dacc.at[tidx_v], add=True)
    plsc.subcore_barrier()
    pltpu.sync_copy(dacc.at[pl.ds(sid * RPT, RPT)],
                    out_hbm.at[cid, pl.ds(sid * RPT, RPT)])


@functools.partial(
    pl.kernel,
    mesh=_mesh,
    out_type=jax.ShapeDtypeStruct((NC, ACC_R, D), jnp.float32),
    scratch_types=[
        pltpu.VMEM((NI, K), jnp.int32),
        pltpu.VMEM((NI, K), jnp.int32),
        pltpu.VMEM((16,), jnp.int32),
        pltpu.VMEM((16,), jnp.int32),
        pltpu.VMEM((NB, K, D), jnp.float32),
        pltpu.VMEM_SHARED((ACC_R, D), jnp.float32),
    ] + [pltpu.SemaphoreType.DMA] * (2 * NI + 2 * NB),
)
def _sc_scatter(g_hbm, ei_hbm, out_hbm,
                sidx_v, didx_v, tsrc_v, tdst_v, rows_v, acc,
                sa0, sa1, sa2, sa3, sa4, sa5,
                sb0, sb1, sb2, sb3, sb4, sb5,
                sg0, sg1, ss0, ss1):
    sa = (sa0, sa1, sa2, sa3, sa4, sa5)
    sb = (sb0, sb1, sb2, sb3, sb4, sb5)
    sg = (sg0, sg1)
    ss = (ss0, ss1)
    cid = lax.axis_index("c")
    sid = lax.axis_index("s")
    wid = sid * NC + cid

    for b in range(NI):
        pltpu.async_copy(ei_hbm.at[0, wid, pl.ds(b * K, K)],
                         sidx_v.at[b], sa[b])
        pltpu.async_copy(ei_hbm.at[1, wid, pl.ds(b * K, K)],
                         didx_v.at[b], sb[b])

    @pl.when(cid == 0)
    def _():
        @pl.when(sid < NS - 1)
        def _():
            pltpu.sync_copy(g_hbm.at[pl.ds(sid * RPT, RPT)],
                            acc.at[pl.ds(sid * RPT, RPT)])

        @pl.when(sid == NS - 1)
        def _():
            pltpu.sync_copy(g_hbm.at[pl.ds((NS - 1) * RPT, G_TAIL)],
                            acc.at[pl.ds((NS - 1) * RPT, G_TAIL)])
            pltpu.sync_copy(g_hbm.at[pl.ds(0, ACC_R - N)],
                            acc.at[pl.ds(N, ACC_R - N)])

    @pl.when(cid == 1)
    def _():
        def zrow(i, carry):
            for c in range(D // 16):
                rows_v[0, i, pl.ds(16 * c, 16)] = jnp.zeros((16,), jnp.float32)
            return carry

        lax.fori_loop(0, K, zrow, 0)
        for r in range(RPT // K):
            pltpu.sync_copy(rows_v.at[0],
                            acc.at[pl.ds(sid * RPT + r * K, K)])

    plsc.subcore_barrier()

    def body(t, carry):
        j0 = NI * t
        for u in range(NI):
            j = j0 + u
            rb = u % NB
            pb = (u - 1) % NI
            qb = (u - 2) % NI
            pltpu.make_async_copy(ei_hbm.at[0, wid, pl.ds(0, K)],
                                  sidx_v.at[u], sa[u]).wait()
            pltpu.make_async_copy(ei_hbm.at[1, wid, pl.ds(0, K)],
                                  didx_v.at[u], sb[u]).wait()

            def wait_prev_scatter():
                pltpu.make_async_copy(rows_v.at[rb],
                                      acc.at[didx_v.at[qb]], ss[rb]).wait()

            if u < 2:
                pl.when(t > 0)(wait_prev_scatter)
            else:
                wait_prev_scatter()
            pltpu.async_copy(g_hbm.at[sidx_v.at[u]], rows_v.at[rb], sg[rb])

            def service_prev():
                pltpu.make_async_copy(g_hbm.at[sidx_v.at[pb]],
                                      rows_v.at[1 - rb], sg[1 - rb]).wait()
                pltpu.async_copy(rows_v.at[1 - rb],
                                 acc.at[didx_v.at[pb]], ss[1 - rb], add=True)

            if u == 0:
                pl.when(t > 0)(service_prev)
            else:
                service_prev()

            @pl.when(jnp.logical_and(j >= 2, j + NI - 2 < CHF))
            def _():
                pltpu.async_copy(ei_hbm.at[0, wid, pl.ds((j + NI - 2) * K, K)],
                                 sidx_v.at[qb], sa[qb])
                pltpu.async_copy(ei_hbm.at[1, wid, pl.ds((j + NI - 2) * K, K)],
                                 didx_v.at[qb], sb[qb])
        return carry

    lax.fori_loop(0, CHF // NI, body, 0)
    lb = (CHF - 1) % NB
    pltpu.make_async_copy(g_hbm.at[sidx_v.at[NI - 1]],
                          rows_v.at[lb], sg[lb]).wait()
    pltpu.async_copy(rows_v.at[lb], acc.at[didx_v.at[NI - 1]], ss[lb],
                     add=True)
    pltpu.make_async_copy(rows_v.at[1 - lb],
                          acc.at[didx_v.at[NI - 2]], ss[1 - lb]).wait()
    pltpu.make_async_copy(rows_v.at[lb],
                          acc.at[didx_v.at[NI - 1]], ss[lb]).wait()
    pltpu.sync_copy(ei_hbm.at[0, wid, pl.ds(CHF * K, TAIL)], tsrc_v)
    pltpu.sync_copy(ei_hbm.at[1, wid, pl.ds(CHF * K, TAIL)], tdst_v)
    pltpu.async_copy(g_hbm.at[tsrc_v], rows_v.at[0, pl.ds(0, TAIL)],
                     sg[0]).wait()
    pltpu.sync_copy(rows_v.at[0, pl.ds(0, TAIL)], acc.at[tdst_v], add=True)
    plsc.subcore_barrier()
    pltpu.sync_copy(acc.at[pl.ds(sid * RPT, RPT)],
                    out_hbm.at[cid, pl.ds(sid * RPT, RPT)])


def _mm_body(x_ref, w_ref, o_ref):
    o_ref[...] = jnp.dot(x_ref[...], w_ref[...],
                         preferred_element_type=jnp.float32)


def _tc_matmul(x, w):
    return pl.pallas_call(
        _mm_body,
        out_shape=jax.ShapeDtypeStruct((x.shape[0], w.shape[1]), jnp.float32),
    )(x, w)


def _scale_body(p_ref, dinv_ref, o_ref):
    o_ref[...] = p_ref[...] * dinv_ref[...]


def _tc_scale(p, dinv):
    return pl.pallas_call(
        _scale_body,
        out_shape=jax.ShapeDtypeStruct(p.shape, jnp.float32),
    )(p, dinv)


def _mid_body(s_ref, dinv_ref, b_ref, w_ref, o_ref):
    s = s_ref[0, :N, :] + s_ref[1, :N, :]
    h = jnp.maximum(dinv_ref[...] * s + b_ref[...], 0.0)
    o_ref[...] = dinv_ref[...] * jnp.dot(h, w_ref[...],
                                         preferred_element_type=jnp.float32)


def _tc_mid(S, dinv, b, w):
    return pl.pallas_call(
        _mid_body,
        out_shape=jax.ShapeDtypeStruct((N, D), jnp.float32),
    )(S, dinv, b, w)


def _out_body(s_ref, dinv_ref, b_ref, wfc_ref, bfc_ref, o_ref):
    s = s_ref[0, :N, :] + s_ref[1, :N, :]
    h = jnp.maximum(dinv_ref[...] * s + b_ref[...], 0.0)
    logit = jnp.dot(h, wfc_ref[...],
                    preferred_element_type=jnp.float32) + bfc_ref[...]
    m = jnp.max(logit, axis=1, keepdims=True)
    ssum = jnp.sum(jnp.exp(logit - m), axis=1, keepdims=True)
    o_ref[...] = logit - m - jnp.log(ssum)


def _tc_out(S, dinv, b, wfc, bfc):
    return pl.pallas_call(
        _out_body,
        out_shape=jax.ShapeDtypeStruct((N, 2), jnp.float32),
    )(S, dinv, b, wfc, bfc)


def kernel(x, edge_index, W1, b1, W2, b2, Wfc, bfc):
    ei = edge_index.reshape(2, NW, EPW)

    degp = _sc_degree(ei)
    p1 = _tc_matmul(x, W1)
    deg = degp[0, :N] + degp[1, :N]
    dinv = lax.rsqrt(deg).reshape(N, 1)
    g1 = _tc_scale(p1, dinv)
    S1 = _sc_scatter(g1, ei)
    g2 = _tc_mid(S1, dinv, b1.reshape(1, D), W2)
    S2 = _sc_scatter(g2, ei)
    return _tc_out(S2, dinv, b2.reshape(1, D), Wfc, bfc.reshape(1, 2))


# speedup vs baseline: 34.9214x; 1.0138x over previous
import functools

import jax
import jax.numpy as jnp
from jax import lax
from jax.experimental import pallas as pl
from jax.experimental.pallas import tpu as pltpu
from jax.experimental.pallas import tpu_sc as plsc

N = 10000
D = 128
E = 320000
NC = 2
NS = 16
NW = NC * NS
EPW = E // NW
K = 128
CHF = EPW // K
TAIL = EPW - CHF * K
NB = 2
NI = 6
ACC_R = 10240
RPT = ACC_R // NS
G_TAIL = N - (NS - 1) * RPT

_mesh = plsc.VectorSubcoreMesh(core_axis_name="c", subcore_axis_name="s")


@functools.partial(
    pl.kernel,
    mesh=_mesh,
    out_type=jax.ShapeDtypeStruct((NC, ACC_R), jnp.float32),
    scratch_types=[
        pltpu.VMEM((3, K), jnp.int32),
        pltpu.VMEM((16,), jnp.int32),
        pltpu.VMEM((K,), jnp.float32),
        pltpu.VMEM((640,), jnp.float32),
        pltpu.VMEM_SHARED((ACC_R,), jnp.float32),
    ] + [pltpu.SemaphoreType.DMA] * 3,
)
def _sc_degree(ei_hbm, out_hbm, idx_v, tidx_v, ones_v, init_v, dacc,
               si0, si1, si2):
    si = (si0, si1, si2)
    cid = lax.axis_index("c")
    sid = lax.axis_index("s")
    wid = sid * NC + cid
    for b in range(3):
        pltpu.async_copy(ei_hbm.at[1, wid, pl.ds(b * K, K)],
                         idx_v.at[b], si[b])
    one = jnp.float32(1.0)
    for i in range(K // 16):
        ones_v[pl.ds(16 * i, 16)] = jnp.full((16,), one, jnp.float32)
    init = jnp.where(cid == 0, one, jnp.float32(0.0))
    for i in range(640 // 16):
        init_v[pl.ds(16 * i, 16)] = jnp.full((16,), init, jnp.float32)
    pltpu.sync_copy(init_v.at[pl.ds(0, RPT)], dacc.at[pl.ds(sid * RPT, RPT)])
    plsc.subcore_barrier()

    def body(t, carry):
        j0 = 3 * t
        for u in range(3):
            j = j0 + u
            pltpu.make_async_copy(ei_hbm.at[1, wid, pl.ds(0, K)],
                                  idx_v.at[u], si[u]).wait()
            pltpu.sync_copy(ones_v, dacc.at[idx_v.at[u]], add=True)

            @pl.when(j + 3 < CHF)
            def _():
                pltpu.async_copy(ei_hbm.at[1, wid, pl.ds((j + 3) * K, K)],
                                 idx_v.at[u], si[u])
        return carry

    lax.fori_loop(0, CHF // 3, body, 0)
    pltpu.sync_copy(ei_hbm.at[1, wid, pl.ds(CHF * K, TAIL)], tidx_v)
    pltpu.sync_copy(ones_v.at[pl.ds(0, TAIL)], dacc.at[tidx_v], add=True)
    plsc.subcore_barrier()
    pltpu.sync_copy(dacc.at[pl.ds(sid * RPT, RPT)],
                    out_hbm.at[cid, pl.ds(sid * RPT, RPT)])


@functools.partial(
    pl.kernel,
    mesh=_mesh,
    out_type=jax.ShapeDtypeStruct((NC, ACC_R, D), jnp.float32),
    scratch_types=[
        pltpu.VMEM((NI, K), jnp.int32),
        pltpu.VMEM((NI, K), jnp.int32),
        pltpu.VMEM((16,), jnp.int32),
        pltpu.VMEM((16,), jnp.int32),
        pltpu.VMEM((NB, K, D), jnp.float32),
        pltpu.VMEM_SHARED((ACC_R, D), jnp.float32),
    ] + [pltpu.SemaphoreType.DMA] * (2 * NI + 2 * NB),
)
def _sc_scatter(g_hbm, ei_hbm, out_hbm,
                sidx_v, didx_v, tsrc_v, tdst_v, rows_v, acc,
                sa0, sa1, sa2, sa3, sa4, sa5,
                sb0, sb1, sb2, sb3, sb4, sb5,
                sg0, sg1, ss0, ss1):
    sa = (sa0, sa1, sa2, sa3, sa4, sa5)
    sb = (sb0, sb1, sb2, sb3, sb4, sb5)
    sg = (sg0, sg1)
    ss = (ss0, ss1)
    cid = lax.axis_index("c")
    sid = lax.axis_index("s")
    wid = sid * NC + cid

    for b in range(NI):
        pltpu.async_copy(ei_hbm.at[0, wid, pl.ds(b * K, K)],
                         sidx_v.at[b], sa[b])
        pltpu.async_copy(ei_hbm.at[1, wid, pl.ds(b * K, K)],
                         didx_v.at[b], sb[b])

    @pl.when(cid == 0)
    def _():
        @pl.when(sid < NS - 1)
        def _():
            pltpu.sync_copy(g_hbm.at[pl.ds(sid * RPT, RPT)],
                            acc.at[pl.ds(sid * RPT, RPT)])

        @pl.when(sid == NS - 1)
        def _():
            pltpu.sync_copy(g_hbm.at[pl.ds((NS - 1) * RPT, G_TAIL)],
                            acc.at[pl.ds((NS - 1) * RPT, G_TAIL)])
            pltpu.sync_copy(g_hbm.at[pl.ds(0, ACC_R - N)],
                            acc.at[pl.ds(N, ACC_R - N)])

    @pl.when(cid == 1)
    def _():
        def zrow(i, carry):
            for c in range(D // 16):
                rows_v[0, i, pl.ds(16 * c, 16)] = jnp.zeros((16,), jnp.float32)
            return carry

        lax.fori_loop(0, K, zrow, 0)
        for r in range(RPT // K):
            pltpu.sync_copy(rows_v.at[0],
                            acc.at[pl.ds(sid * RPT + r * K, K)])

    plsc.subcore_barrier()

    def body(t, carry):
        j0 = NI * t
        for u in range(NI):
            j = j0 + u
            rb = u % NB
            pb = (u - 1) % NI
            qb = (u - 2) % NI
            pltpu.make_async_copy(ei_hbm.at[0, wid, pl.ds(0, K)],
                                  sidx_v.at[u], sa[u]).wait()
            pltpu.make_async_copy(ei_hbm.at[1, wid, pl.ds(0, K)],
                                  didx_v.at[u], sb[u]).wait()

            def wait_prev_scatter():
                pltpu.make_async_copy(rows_v.at[rb],
                                      acc.at[didx_v.at[qb]], ss[rb]).wait()

            if u < 2:
                pl.when(t > 0)(wait_prev_scatter)
            else:
                wait_prev_scatter()
            pltpu.async_copy(g_hbm.at[sidx_v.at[u]], rows_v.at[rb], sg[rb])

            def service_prev():
                pltpu.make_async_copy(g_hbm.at[sidx_v.at[pb]],
                                      rows_v.at[1 - rb], sg[1 - rb]).wait()
                pltpu.async_copy(rows_v.at[1 - rb],
                                 acc.at[didx_v.at[pb]], ss[1 - rb], add=True)

            if u == 0:
                pl.when(t > 0)(service_prev)
            else:
                service_prev()

            @pl.when(jnp.logical_and(j >= 2, j + NI - 2 < CHF))
            def _():
                pltpu.async_copy(ei_hbm.at[0, wid, pl.ds((j + NI - 2) * K, K)],
                                 sidx_v.at[qb], sa[qb])
                pltpu.async_copy(ei_hbm.at[1, wid, pl.ds((j + NI - 2) * K, K)],
                                 didx_v.at[qb], sb[qb])
        return carry

    lax.fori_loop(0, CHF // NI, body, 0)
    lb = (CHF - 1) % NB
    pltpu.make_async_copy(g_hbm.at[sidx_v.at[NI - 1]],
                          rows_v.at[lb], sg[lb]).wait()
    pltpu.async_copy(rows_v.at[lb], acc.at[didx_v.at[NI - 1]], ss[lb],
                     add=True)
    pltpu.make_async_copy(rows_v.at[1 - lb],
                          acc.at[didx_v.at[NI - 2]], ss[1 - lb]).wait()
    pltpu.make_async_copy(rows_v.at[lb],
                          acc.at[didx_v.at[NI - 1]], ss[lb]).wait()
    pltpu.sync_copy(ei_hbm.at[0, wid, pl.ds(CHF * K, TAIL)], tsrc_v)
    pltpu.sync_copy(ei_hbm.at[1, wid, pl.ds(CHF * K, TAIL)], tdst_v)
    pltpu.async_copy(g_hbm.at[tsrc_v], rows_v.at[0, pl.ds(0, TAIL)],
                     sg[0]).wait()
    pltpu.sync_copy(rows_v.at[0, pl.ds(0, TAIL)], acc.at[tdst_v], add=True)
    plsc.subcore_barrier()
    pltpu.sync_copy(acc.at[pl.ds(sid * RPT, RPT)],
                    out_hbm.at[cid, pl.ds(sid * RPT, RPT)])


def _mm_body(x_ref, w_ref, o_ref):
    o_ref[...] = jnp.dot(x_ref[...], w_ref[...],
                         preferred_element_type=jnp.float32)


def _tc_matmul(x, w):
    return pl.pallas_call(
        _mm_body,
        out_shape=jax.ShapeDtypeStruct((x.shape[0], w.shape[1]), jnp.float32),
    )(x, w)


def _scale_body(p_ref, degp_ref, o_ref, dinv_ref):
    deg = degp_ref[0:1, :N] + degp_ref[1:2, :N]
    dinv = jnp.transpose(lax.rsqrt(deg), (1, 0))
    dinv_ref[...] = dinv
    o_ref[...] = p_ref[...] * dinv


def _tc_scale(p, degp):
    return pl.pallas_call(
        _scale_body,
        out_shape=(jax.ShapeDtypeStruct(p.shape, jnp.float32),
                   jax.ShapeDtypeStruct((N, 1), jnp.float32)),
    )(p, degp)


def _mid_body(s_ref, dinv_ref, b_ref, w_ref, o_ref):
    s = s_ref[0, :N, :] + s_ref[1, :N, :]
    h = jnp.maximum(dinv_ref[...] * s + b_ref[...], 0.0)
    o_ref[...] = dinv_ref[...] * jnp.dot(h, w_ref[...],
                                         preferred_element_type=jnp.float32)


def _tc_mid(S, dinv, b, w):
    return pl.pallas_call(
        _mid_body,
        out_shape=jax.ShapeDtypeStruct((N, D), jnp.float32),
    )(S, dinv, b, w)


def _out_body(s_ref, dinv_ref, b_ref, wfc_ref, bfc_ref, o_ref):
    s = s_ref[0, :N, :] + s_ref[1, :N, :]
    h = jnp.maximum(dinv_ref[...] * s + b_ref[...], 0.0)
    logit = jnp.dot(h, wfc_ref[...],
                    preferred_element_type=jnp.float32) + bfc_ref[...]
    m = jnp.max(logit, axis=1, keepdims=True)
    ssum = jnp.sum(jnp.exp(logit - m), axis=1, keepdims=True)
    o_ref[...] = logit - m - jnp.log(ssum)


def _tc_out(S, dinv, b, wfc, bfc):
    return pl.pallas_call(
        _out_body,
        out_shape=jax.ShapeDtypeStruct((N, 2), jnp.float32),
    )(S, dinv, b, wfc, bfc)


def kernel(x, edge_index, W1, b1, W2, b2, Wfc, bfc):
    ei = edge_index.reshape(2, NW, EPW)

    degp = _sc_degree(ei)
    p1 = _tc_matmul(x, W1)
    g1, dinv = _tc_scale(p1, degp)
    S1 = _sc_scatter(g1, ei)
    g2 = _tc_mid(S1, dinv, b1.reshape(1, D), W2)
    S2 = _sc_scatter(g2, ei)
    return _tc_out(S2, dinv, b2.reshape(1, D), Wfc, bfc.reshape(1, 2))
